# baseline (speedup 1.0000x reference)
import functools

import jax
import jax.numpy as jnp
from jax import lax
from jax.experimental import pallas as pl
from jax.experimental.pallas import tpu as pltpu
from jax.experimental.pallas import tpu_sc as plsc

_F32 = jnp.float32
_BF16 = jnp.bfloat16


def _mmb(a, b):
    return jnp.dot(a.astype(_BF16), b.astype(_BF16),
                   preferred_element_type=_F32)


def _round8(n):
    return ((n + 7) // 8) * 8


def _sub_tree(v):
    v = v[0:4] + v[4:8]
    v = v[0:2] + v[2:4]
    return v[0:1] + v[1:2]


def _group_sum(Xq):
    S = Xq.shape[0]
    if S == 32:
        acc = None
        for j in range(2):
            for i in range(4):
                t = Xq[8 * i:8 * i + 8, 128 * j:128 * j + 128]
                acc = t if acc is None else acc + t
    else:
        acc = Xq[:, 0:128] + Xq[:, 128:256]
    return jnp.sum(_sub_tree(acc), axis=1, keepdims=True)


def _group_norm(X, gamma, beta, eps=1e-5):
    L, C = X.shape
    S = C // 8
    n = float(L * S)
    XT = X.T
    mparts, vparts = [], []
    for g in range(8):
        Xq = XT[g * S:(g + 1) * S]
        m = _group_sum(Xq) / n
        xc = Xq - m
        v = _group_sum(xc * xc) / n
        mparts.append(jnp.broadcast_to(m, (1, S)))
        vparts.append(jnp.broadcast_to(v, (1, S)))
    mrow = jnp.concatenate(mparts, axis=1)
    vrow = jnp.concatenate(vparts, axis=1)
    Xn = (X - mrow) * lax.rsqrt(vrow + eps)
    return Xn * gamma + beta


def _conv_stage(x, w_stack, bias, offsets, relu):
    B, L, Cin = x.shape
    K, _, Cout = w_stack.shape
    pad_lo = max(0, -min(offsets))
    ext = _round8(L + pad_lo + max(0, max(offsets)))

    def body(x_ref, w_ref, b_ref, o_ref, xp_ref):
        xp_ref[...] = jnp.zeros((ext, Cin), _F32)
        xp_ref[pad_lo:pad_lo + L] = x_ref[0]
        acc = None
        for k, o in enumerate(offsets):
            t = _mmb(xp_ref[pad_lo + o: pad_lo + o + L], w_ref[k])
            acc = t if acc is None else acc + t
        acc = acc + b_ref[...]
        if relu:
            acc = jnp.maximum(acc, 0.0)
        o_ref[0] = acc

    return pl.pallas_call(
        body,
        grid=(B,),
        in_specs=[
            pl.BlockSpec((1, L, Cin), lambda b: (b, 0, 0)),
            pl.BlockSpec((K, Cin, Cout), lambda b: (0, 0, 0)),
            pl.BlockSpec((1, Cout), lambda b: (0, 0)),
        ],
        out_specs=pl.BlockSpec((1, L, Cout), lambda b: (b, 0, 0)),
        out_shape=jax.ShapeDtypeStruct((B, L, Cout), _F32),
        scratch_shapes=[pltpu.VMEM((ext, Cin), _F32)],
    )(x, w_stack, bias)


def _enc2_stage(he, ho, w_taps, bias):
    B, L, Cin = he.shape
    Cout = w_taps.shape[2]
    ext = _round8(L + 2)

    def body(he_ref, ho_ref, w_ref, b_ref, o_ref, hep_ref, hop_ref):
        hep_ref[...] = jnp.zeros((ext, Cin), _F32)
        hop_ref[...] = jnp.zeros((ext, Cin), _F32)
        hep_ref[1:1 + L] = he_ref[0]
        hop_ref[1:1 + L] = ho_ref[0]
        acc = _mmb(hep_ref[0:L], w_ref[0])
        acc = acc + _mmb(hop_ref[0:L], w_ref[1])
        acc = acc + _mmb(hep_ref[1:1 + L], w_ref[2])
        acc = acc + _mmb(hop_ref[1:1 + L], w_ref[3])
        acc = acc + _mmb(hep_ref[2:2 + L], w_ref[4])
        o_ref[0] = jnp.maximum(acc + b_ref[...], 0.0)

    return pl.pallas_call(
        body,
        grid=(B,),
        in_specs=[
            pl.BlockSpec((1, L, Cin), lambda b: (b, 0, 0)),
            pl.BlockSpec((1, L, Cin), lambda b: (b, 0, 0)),
            pl.BlockSpec((5, Cin, Cout), lambda b: (0, 0, 0)),
            pl.BlockSpec((1, Cout), lambda b: (0, 0)),
        ],
        out_specs=pl.BlockSpec((1, L, Cout), lambda b: (b, 0, 0)),
        out_shape=jax.ShapeDtypeStruct((B, L, Cout), _F32),
        scratch_shapes=[pltpu.VMEM((ext, Cin), _F32),
                        pltpu.VMEM((ext, Cin), _F32)],
    )(he, ho, w_taps, bias)


def _vq_encoder_stage(he2, ho2, w3_taps, b3, res_params, wpre, bpre, embT):
    B, L, Cin = he2.shape
    C = 256
    D, V = embT.shape
    ext = _round8(L + 2)

    def body(he_ref, ho_ref, w3_ref, b3_ref,
             r0g1g, r0g1b, r0c1, r0g2g, r0g2b, r0c2,
             r1g1g, r1g1b, r1c1, r1g2g, r1g2b, r1c2,
             wpre_ref, bpre_ref, embT_ref,
             z_ref, idx_ref, cnt_ref, sp_ref, pp_ref):
        b = pl.program_id(0)
        pp_ref[...] = jnp.zeros((ext, Cin), _F32)
        pp_ref[1:1 + L] = ho_ref[0]
        h = _mmb(pp_ref[0:L], w3_ref[0])
        h = h + _mmb(he_ref[0], w3_ref[1])
        h = h + _mmb(pp_ref[1:1 + L], w3_ref[2])
        h = h + b3_ref[...]
        for (g1g, g1b, c1, g2g, g2b, c2) in (
                (r0g1g, r0g1b, r0c1, r0g2g, r0g2b, r0c2),
                (r1g1g, r1g1b, r1c1, r1g2g, r1g2b, r1c2)):
            r = _group_norm(h, g1g[...], g1b[...])
            r = jnp.maximum(r, 0.0)
            sp_ref[...] = jnp.zeros((ext, C), _F32)
            sp_ref[1:1 + L] = r
            r2 = _mmb(sp_ref[0:L], c1[0])
            r2 = r2 + _mmb(sp_ref[1:1 + L], c1[1])
            r2 = r2 + _mmb(sp_ref[2:2 + L], c1[2])
            r2 = _group_norm(r2, g2g[...], g2b[...])
            r2 = jnp.maximum(r2, 0.0)
            h = h + _mmb(r2, c2[...])
        z = _mmb(h, wpre_ref[...]) + bpre_ref[...]
        z_ref[0] = z
        eT = embT_ref[...]
        sq = eT * eT
        acc = None
        for i in range(8):
            t = sq[8 * i:8 * i + 8]
            acc = t if acc is None else acc + t
        e2 = _sub_tree(acc)
        zz = jnp.sum(z * z, axis=1, keepdims=True)
        dist = zz + e2 - 2.0 * _mmb(z, eT)
        mind = jnp.min(dist, axis=1, keepdims=True)
        li = lax.broadcasted_iota(jnp.int32, (L, V), 1)
        idxm = jnp.min(jnp.where(dist <= mind, li, jnp.int32(2 ** 30)),
                       axis=1, keepdims=True)
        idx_ref[0] = idxm
        oh = (li == idxm).astype(_F32)
        cnt = jnp.sum(oh, axis=0, keepdims=True)

        @pl.when(b == 0)
        def _():
            cnt_ref[...] = jnp.zeros((1, V), _F32)

        cnt_ref[...] = cnt_ref[...] + cnt

    full = lambda *s: pl.BlockSpec(s, lambda b: (0,) * len(s))
    in_specs = [pl.BlockSpec((1, L, Cin), lambda b: (b, 0, 0)),
                pl.BlockSpec((1, L, Cin), lambda b: (b, 0, 0)),
                full(3, Cin, C), full(1, C)]
    for _ in range(2):
        in_specs += [full(1, 256), full(1, 256), full(3, 256, 64),
                     full(1, 64), full(1, 64), full(64, 256)]
    in_specs += [full(C, D), full(1, D), full(D, V)]

    return pl.pallas_call(
        body,
        grid=(B,),
        in_specs=in_specs,
        out_specs=[
            pl.BlockSpec((1, L, D), lambda b: (b, 0, 0)),
            pl.BlockSpec((1, L, 1), lambda b: (b, 0, 0)),
            pl.BlockSpec((1, V), lambda b: (0, 0)),
        ],
        out_shape=[
            jax.ShapeDtypeStruct((B, L, D), _F32),
            jax.ShapeDtypeStruct((B, L, 1), jnp.int32),
            jax.ShapeDtypeStruct((1, V), _F32),
        ],
        scratch_shapes=[pltpu.VMEM((ext, C), _F32),
                        pltpu.VMEM((ext, Cin), _F32)],
    )(he2, ho2, w3_taps, b3, *res_params, wpre, bpre, embT)


def _sc_codebook_gather(table, idx):
    N = idx.shape[0]
    V, D = table.shape
    info = plsc.get_sparse_core_info()
    NC, NS = info.num_cores, info.num_subcores
    NW = NC * NS
    bpw = N // NW

    @functools.partial(
        pl.kernel,
        out_type=jax.ShapeDtypeStruct((N, D), _F32),
        mesh=plsc.VectorSubcoreMesh(core_axis_name="c", subcore_axis_name="s"),
        scratch_types=[
            pltpu.VMEM((bpw,), jnp.int32),
            pltpu.VMEM((bpw, D), _F32),
            pltpu.SemaphoreType.DMA,
        ],
    )
    def gather_kernel(table_hbm, idx_hbm, out_hbm, idx_v, rows_v, sem):
        wid = lax.axis_index("s") * NC + lax.axis_index("c")
        base = wid * bpw
        pltpu.sync_copy(idx_hbm.at[pl.ds(base, bpw)], idx_v)
        pltpu.async_copy(table_hbm.at[idx_v], rows_v, sem).wait()
        pltpu.sync_copy(rows_v, out_hbm.at[pl.ds(base, bpw)])

    return gather_kernel(table, idx)


def _dec1_stage(q, z, p_stack, bias_pair):
    B, L, D = q.shape
    Co = p_stack.shape[2]
    ext = _round8(L + 1)

    def body(q_ref, z_ref, p_ref, bp_ref, y_ref, sq_ref, pool_ref, qp_ref):
        b = pl.program_id(0)
        d = q_ref[0] - z_ref[0]
        q_st = z_ref[0] + d
        qp_ref[...] = jnp.zeros((ext, D), _F32)
        qp_ref[0:L] = q_st
        y = _mmb(q_st, p_ref[0]) + _mmb(qp_ref[1:1 + L], p_ref[1]) + bp_ref[...]
        y_ref[0] = jnp.maximum(y, 0.0)

        @pl.when(b == 0)
        def _():
            sq_ref[...] = jnp.zeros((1, 1), _F32)

        sq_ref[...] = sq_ref[...] + jnp.sum(d * d)
        pool_ref[0] = jnp.sum(q_st, axis=0, keepdims=True) / L

    return pl.pallas_call(
        body,
        grid=(B,),
        in_specs=[
            pl.BlockSpec((1, L, D), lambda b: (b, 0, 0)),
            pl.BlockSpec((1, L, D), lambda b: (b, 0, 0)),
            pl.BlockSpec((2, D, Co), lambda b: (0, 0, 0)),
            pl.BlockSpec((1, Co), lambda b: (0, 0)),
        ],
        out_specs=[
            pl.BlockSpec((1, L, Co), lambda b: (b, 0, 0)),
            pl.BlockSpec((1, 1), lambda b: (0, 0)),
            pl.BlockSpec((1, 1, D), lambda b: (b, 0, 0)),
        ],
        out_shape=[
            jax.ShapeDtypeStruct((B, L, Co), _F32),
            jax.ShapeDtypeStruct((1, 1), _F32),
            jax.ShapeDtypeStruct((B, 1, D), _F32),
        ],
        scratch_shapes=[pltpu.VMEM((ext, D), _F32)],
    )(q, z, p_stack, bias_pair)


def _head_stage(counts, sq, pooled, w1t, b1, w2t, b2, w3t, b3, n_tok, n_lat):
    Bp, D = pooled.shape
    V = counts.shape[1]

    def body(cnt_ref, sq_ref, pool_ref, w1_ref, b1_ref, w2_ref, b2_ref,
             w3_ref, b3_ref, loss_ref, perp_ref, bp_ref):
        probs = cnt_ref[...] / n_tok
        perp_ref[...] = jnp.exp(-jnp.sum(probs * jnp.log(probs + 1e-10))
                                ) * jnp.ones((1, 1), _F32)
        loss_ref[...] = 0.25 * sq_ref[...] / n_lat
        h = jnp.maximum(_mmb(pool_ref[...], w1_ref[...]) + b1_ref[...], 0.0)
        h = jnp.maximum(_mmb(h, w2_ref[...]) + b2_ref[...], 0.0)
        bp_ref[...] = _mmb(h, w3_ref[...]) + b3_ref[...]

    return pl.pallas_call(
        body,
        out_shape=[
            jax.ShapeDtypeStruct((1, 1), _F32),
            jax.ShapeDtypeStruct((1, 1), _F32),
            jax.ShapeDtypeStruct((Bp, 4), _F32),
        ],
    )(counts, sq, pooled, w1t, b1, w2t, b2, w3t, b3)


def kernel(x, params):
    p = params
    B, Cx, Lx = x.shape
    xt = jnp.transpose(x, (0, 2, 1))

    W1 = jnp.transpose(p['enc_c1_w'], (2, 1, 0))
    b1 = p['enc_c1_b'][None]
    W2t = jnp.transpose(p['enc_c2_w'], (2, 1, 0))
    b2 = p['enc_c2_b'][None]
    W3t = jnp.transpose(p['enc_c3_w'], (2, 1, 0))
    b3 = p['enc_c3_b'][None]
    res_params = []
    for i in range(2):
        res_params += [
            p['res%d_gn1_g' % i][None], p['res%d_gn1_b' % i][None],
            jnp.transpose(p['res%d_c1_w' % i], (2, 1, 0)),
            p['res%d_gn2_g' % i][None], p['res%d_gn2_b' % i][None],
            p['res%d_c2_w' % i][:, :, 0].T,
        ]
    wpre = p['prevq_w'][:, :, 0].T
    bpre = p['prevq_b'][None]
    emb = p['codebook']
    embT = emb.T
    w_ct1 = p['dec_ct1_w']
    z256 = jnp.zeros((64, 256), _F32)
    P_ct1 = jnp.stack([
        jnp.concatenate([w_ct1[:, :, 1], w_ct1[:, :, 2]], 1),
        jnp.concatenate([z256, w_ct1[:, :, 0]], 1)])
    b_ct1 = jnp.concatenate([p['dec_ct1_b'], p['dec_ct1_b']])[None]
    w_ct2 = p['dec_ct2_w']
    z128b = jnp.zeros((256, 128), _F32)
    Q_ct2 = jnp.stack([
        jnp.concatenate([w_ct2[:, :, 4], z128b], 1),
        jnp.concatenate([w_ct2[:, :, 2], w_ct2[:, :, 3]], 1),
        jnp.concatenate([w_ct2[:, :, 0], w_ct2[:, :, 1]], 1)])
    b_ct2 = jnp.concatenate([p['dec_ct2_b'], p['dec_ct2_b']])[None]
    W7 = jnp.transpose(p['dec_c3_w'], (2, 1, 0))
    b7 = p['dec_c3_b'][None]

    h1 = _conv_stage(xt, W1, b1, (-3, -2, -1, 0, 1, 2, 3), relu=True)
    h2 = _enc2_stage(h1[:, 0::2, :], h1[:, 1::2, :], W2t, b2)
    z, idx3, counts = _vq_encoder_stage(
        h2[:, 0::2, :], h2[:, 1::2, :], W3t, b3, res_params, wpre, bpre, embT)
    emb_q = emb.astype(_BF16).astype(_F32)
    emb_pad = jnp.concatenate([emb_q, jnp.zeros((1024, 64), _F32)], axis=1)
    qflat = _sc_codebook_gather(emb_pad, idx3.reshape(B * 256))
    q = qflat[:, 0:64].reshape(B, 256, 64)
    y_pair, sq, pooled3 = _dec1_stage(q, z, P_ct1, b_ct1)
    y2 = _conv_stage(y_pair.reshape(B, 512, 256), Q_ct2, b_ct2, (-1, 0, 1),
                     relu=True)
    xrt = _conv_stage(y2.reshape(B, 1024, 128), W7, b7,
                      (-3, -2, -1, 0, 1, 2, 3), relu=False)
    x_recon = jnp.transpose(xrt, (0, 2, 1))
    loss11, perp11, behavior_pred = _head_stage(
        counts, sq, pooled3.reshape(B, 64),
        p['bh1_w'].T, p['bh1_b'][None], p['bh2_w'].T, p['bh2_b'][None],
        p['bh3_w'].T, p['bh3_b'][None],
        n_tok=float(B * 256), n_lat=float(B * 256 * 64))
    return (loss11.reshape(()), x_recon, perp11.reshape(()),
            behavior_pred)


# speedup vs baseline: 1.3289x; 1.3289x over previous
import functools

import jax
import jax.numpy as jnp
from jax import lax
from jax.experimental import pallas as pl
from jax.experimental.pallas import tpu as pltpu
from jax.experimental.pallas import tpu_sc as plsc

_F32 = jnp.float32
_BF16 = jnp.bfloat16


def _mmb(a, b):
    return jnp.dot(a.astype(_BF16), b.astype(_BF16),
                   preferred_element_type=_F32)


def _round8(n):
    return ((n + 7) // 8) * 8


def _sub_tree(v):
    v = v[0:4] + v[4:8]
    v = v[0:2] + v[2:4]
    return v[0:1] + v[1:2]


def _group_sum(Xq):
    S = Xq.shape[0]
    if S == 32:
        acc = None
        for j in range(2):
            for i in range(4):
                t = Xq[8 * i:8 * i + 8, 128 * j:128 * j + 128]
                acc = t if acc is None else acc + t
    else:
        acc = Xq[:, 0:128] + Xq[:, 128:256]
    return jnp.sum(_sub_tree(acc), axis=1, keepdims=True)


def _group_norm(X, gamma, beta, eps=1e-5):
    L, C = X.shape
    S = C // 8
    n = float(L * S)
    XT = X.T
    mparts, vparts = [], []
    for g in range(8):
        Xq = XT[g * S:(g + 1) * S]
        m = _group_sum(Xq) / n
        xc = Xq - m
        v = _group_sum(xc * xc) / n
        mparts.append(jnp.broadcast_to(m, (1, S)))
        vparts.append(jnp.broadcast_to(v, (1, S)))
    mrow = jnp.concatenate(mparts, axis=1)
    vrow = jnp.concatenate(vparts, axis=1)
    Xn = (X - mrow) * lax.rsqrt(vrow + eps)
    return Xn * gamma + beta


def _conv_stage(x, w_stack, bias, offsets, relu):
    B, L, Cin = x.shape
    K, _, Cout = w_stack.shape
    pad_lo = max(0, -min(offsets))
    ext = _round8(L + pad_lo + max(0, max(offsets)))

    def body(x_ref, w_ref, b_ref, o_ref, xp_ref):
        xp_ref[...] = jnp.zeros((ext, Cin), _F32)
        xp_ref[pad_lo:pad_lo + L] = x_ref[0]
        acc = None
        for k, o in enumerate(offsets):
            t = _mmb(xp_ref[pad_lo + o: pad_lo + o + L], w_ref[k])
            acc = t if acc is None else acc + t
        acc = acc + b_ref[...]
        if relu:
            acc = jnp.maximum(acc, 0.0)
        o_ref[0] = acc

    return pl.pallas_call(
        body,
        grid=(B,),
        in_specs=[
            pl.BlockSpec((1, L, Cin), lambda b: (b, 0, 0)),
            pl.BlockSpec((K, Cin, Cout), lambda b: (0, 0, 0)),
            pl.BlockSpec((1, Cout), lambda b: (0, 0)),
        ],
        out_specs=pl.BlockSpec((1, L, Cout), lambda b: (b, 0, 0)),
        out_shape=jax.ShapeDtypeStruct((B, L, Cout), _F32),
        scratch_shapes=[pltpu.VMEM((ext, Cin), _F32)],
    )(x, w_stack, bias)


def _enc2_stage(hp, w01, w4pad, bias):
    B, L, Cp = hp.shape
    Cout = w01.shape[1]
    ext = _round8(L + 2)

    def body(hp_ref, w01_ref, w4_ref, b_ref, o_ref, hpp_ref):
        hpp_ref[...] = jnp.zeros((ext, Cp), _F32)
        hpp_ref[1:1 + L] = hp_ref[0]
        cat = jnp.concatenate([hpp_ref[0:L], hpp_ref[1:1 + L]], axis=1)
        acc = _mmb(cat, w01_ref[...]) + _mmb(hpp_ref[2:2 + L], w4_ref[...])
        o_ref[0] = jnp.maximum(acc + b_ref[...], 0.0)

    return pl.pallas_call(
        body,
        grid=(B,),
        in_specs=[
            pl.BlockSpec((1, L, Cp), lambda b: (b, 0, 0)),
            pl.BlockSpec((2 * Cp, Cout), lambda b: (0, 0)),
            pl.BlockSpec((Cp, Cout), lambda b: (0, 0)),
            pl.BlockSpec((1, Cout), lambda b: (0, 0)),
        ],
        out_specs=pl.BlockSpec((1, L, Cout), lambda b: (b, 0, 0)),
        out_shape=jax.ShapeDtypeStruct((B, L, Cout), _F32),
        scratch_shapes=[pltpu.VMEM((ext, Cp), _F32)],
    )(hp, w01, w4pad, bias)


def _vq_encoder_stage(hp2, w3_01, w3_2pad, b3, res_params, wpre, bpre, embT):
    B, L, C = hp2.shape
    D, V = embT.shape
    ext = _round8(L + 2)

    def body(hp_ref, w3_ref, w32_ref, b3_ref,
             r0g1g, r0g1b, r0c1, r0g2g, r0g2b, r0c2,
             r1g1g, r1g1b, r1c1, r1g2g, r1g2b, r1c2,
             wpre_ref, bpre_ref, embT_ref,
             z_ref, idx_ref, cnt_ref, sp_ref):
        b = pl.program_id(0)
        sp_ref[...] = jnp.zeros((ext, C), _F32)
        sp_ref[1:1 + L] = hp_ref[0]
        cat = jnp.concatenate([sp_ref[0:L, 128:256], sp_ref[1:1 + L, 0:128]],
                              axis=1)
        h = _mmb(cat, w3_ref[...]) + _mmb(sp_ref[1:1 + L], w32_ref[...])
        h = h + b3_ref[...]
        for (g1g, g1b, c1, g2g, g2b, c2) in (
                (r0g1g, r0g1b, r0c1, r0g2g, r0g2b, r0c2),
                (r1g1g, r1g1b, r1c1, r1g2g, r1g2b, r1c2)):
            r = _group_norm(h, g1g[...], g1b[...])
            r = jnp.maximum(r, 0.0)
            sp_ref[...] = jnp.zeros((ext, C), _F32)
            sp_ref[1:1 + L] = r
            r2 = _mmb(sp_ref[0:L], c1[0])
            r2 = r2 + _mmb(sp_ref[1:1 + L], c1[1])
            r2 = r2 + _mmb(sp_ref[2:2 + L], c1[2])
            r2 = _group_norm(r2, g2g[...], g2b[...])
            r2 = jnp.maximum(r2, 0.0)
            h = h + _mmb(r2, c2[...])
        z = _mmb(h, wpre_ref[...]) + bpre_ref[...]
        z_ref[0] = z
        eT = embT_ref[...]
        sq = eT * eT
        acc = None
        for i in range(8):
            t = sq[8 * i:8 * i + 8]
            acc = t if acc is None else acc + t
        e2 = _sub_tree(acc)
        zz = jnp.sum(z * z, axis=1, keepdims=True)
        dist = zz + e2 - 2.0 * _mmb(z, eT)
        mind = jnp.min(dist, axis=1, keepdims=True)
        li = lax.broadcasted_iota(jnp.int32, (L, V), 1)
        idxm = jnp.min(jnp.where(dist <= mind, li, jnp.int32(2 ** 30)),
                       axis=1, keepdims=True)
        idx_ref[0] = idxm
        oh = (li == idxm).astype(_F32)
        cnt = jnp.sum(oh, axis=0, keepdims=True)

        @pl.when(b == 0)
        def _():
            cnt_ref[...] = jnp.zeros((1, V), _F32)

        cnt_ref[...] = cnt_ref[...] + cnt

    full = lambda *s: pl.BlockSpec(s, lambda b: (0,) * len(s))
    in_specs = [pl.BlockSpec((1, L, C), lambda b: (b, 0, 0)),
                full(C, C), full(C, C), full(1, C)]
    for _ in range(2):
        in_specs += [full(1, 256), full(1, 256), full(3, 256, 64),
                     full(1, 64), full(1, 64), full(64, 256)]
    in_specs += [full(C, D), full(1, D), full(D, V)]

    return pl.pallas_call(
        body,
        grid=(B,),
        in_specs=in_specs,
        out_specs=[
            pl.BlockSpec((1, L, D), lambda b: (b, 0, 0)),
            pl.BlockSpec((1, L, 1), lambda b: (b, 0, 0)),
            pl.BlockSpec((1, V), lambda b: (0, 0)),
        ],
        out_shape=[
            jax.ShapeDtypeStruct((B, L, D), _F32),
            jax.ShapeDtypeStruct((B, L, 1), jnp.int32),
            jax.ShapeDtypeStruct((1, V), _F32),
        ],
        scratch_shapes=[pltpu.VMEM((ext, C), _F32)],
    )(hp2, w3_01, w3_2pad, b3, *res_params, wpre, bpre, embT)


def _sc_codebook_gather(table, idx):
    N = idx.shape[0]
    V, D = table.shape
    info = plsc.get_sparse_core_info()
    NC, NS = info.num_cores, info.num_subcores
    NW = NC * NS
    bpw = N // NW

    @functools.partial(
        pl.kernel,
        out_type=jax.ShapeDtypeStruct((N, D), _F32),
        mesh=plsc.VectorSubcoreMesh(core_axis_name="c", subcore_axis_name="s"),
        compiler_params=pltpu.CompilerParams(use_tc_tiling_on_sc=False),
        scratch_types=[
            pltpu.VMEM((bpw,), jnp.int32),
            pltpu.VMEM((bpw, D), _F32),
            pltpu.SemaphoreType.DMA,
        ],
    )
    def gather_kernel(table_hbm, idx_hbm, out_hbm, idx_v, rows_v, sem):
        wid = lax.axis_index("s") * NC + lax.axis_index("c")
        base = wid * bpw
        pltpu.sync_copy(idx_hbm.at[pl.ds(base, bpw)], idx_v)
        pltpu.async_copy(table_hbm.at[idx_v], rows_v, sem).wait()
        pltpu.sync_copy(rows_v, out_hbm.at[pl.ds(base, bpw)])

    return gather_kernel(table, idx)


def _dec1_stage(q, z, p_stack, bias_pair):
    B, L, D = q.shape
    Co = p_stack.shape[2]
    ext = _round8(L + 1)

    def body(q_ref, z_ref, p_ref, bp_ref, y_ref, sq_ref, pool_ref, qp_ref):
        b = pl.program_id(0)
        d = q_ref[0] - z_ref[0]
        q_st = z_ref[0] + d
        qp_ref[...] = jnp.zeros((ext, D), _F32)
        qp_ref[0:L] = q_st
        y = _mmb(q_st, p_ref[0]) + _mmb(qp_ref[1:1 + L], p_ref[1]) + bp_ref[...]
        y_ref[0] = jnp.maximum(y, 0.0)

        @pl.when(b == 0)
        def _():
            sq_ref[...] = jnp.zeros((1, 1), _F32)

        sq_ref[...] = sq_ref[...] + jnp.sum(d * d)
        pool_ref[0] = jnp.sum(q_st, axis=0, keepdims=True) / L

    return pl.pallas_call(
        body,
        grid=(B,),
        in_specs=[
            pl.BlockSpec((1, L, D), lambda b: (b, 0, 0)),
            pl.BlockSpec((1, L, D), lambda b: (b, 0, 0)),
            pl.BlockSpec((2, D, Co), lambda b: (0, 0, 0)),
            pl.BlockSpec((1, Co), lambda b: (0, 0)),
        ],
        out_specs=[
            pl.BlockSpec((1, L, Co), lambda b: (b, 0, 0)),
            pl.BlockSpec((1, 1), lambda b: (0, 0)),
            pl.BlockSpec((1, 1, D), lambda b: (b, 0, 0)),
        ],
        out_shape=[
            jax.ShapeDtypeStruct((B, L, Co), _F32),
            jax.ShapeDtypeStruct((1, 1), _F32),
            jax.ShapeDtypeStruct((B, 1, D), _F32),
        ],
        scratch_shapes=[pltpu.VMEM((ext, D), _F32)],
    )(q, z, p_stack, bias_pair)


def _head_stage(counts, sq, pooled, w1t, b1, w2t, b2, w3t, b3, n_tok, n_lat):
    Bp, D = pooled.shape
    V = counts.shape[1]

    def body(cnt_ref, sq_ref, pool_ref, w1_ref, b1_ref, w2_ref, b2_ref,
             w3_ref, b3_ref, loss_ref, perp_ref, bp_ref):
        probs = cnt_ref[...] / n_tok
        perp_ref[...] = jnp.exp(-jnp.sum(probs * jnp.log(probs + 1e-10))
                                ) * jnp.ones((1, 1), _F32)
        loss_ref[...] = 0.25 * sq_ref[...] / n_lat
        h = jnp.maximum(_mmb(pool_ref[...], w1_ref[...]) + b1_ref[...], 0.0)
        h = jnp.maximum(_mmb(h, w2_ref[...]) + b2_ref[...], 0.0)
        bp_ref[...] = _mmb(h, w3_ref[...]) + b3_ref[...]

    return pl.pallas_call(
        body,
        out_shape=[
            jax.ShapeDtypeStruct((1, 1), _F32),
            jax.ShapeDtypeStruct((1, 1), _F32),
            jax.ShapeDtypeStruct((Bp, 4), _F32),
        ],
    )(counts, sq, pooled, w1t, b1, w2t, b2, w3t, b3)


def kernel(x, params):
    p = params
    B, Cx, Lx = x.shape
    xt = jnp.transpose(x, (0, 2, 1))

    W1 = jnp.transpose(p['enc_c1_w'], (2, 1, 0))
    b1 = p['enc_c1_b'][None]
    W2t = jnp.transpose(p['enc_c2_w'], (2, 1, 0))
    W2_01 = jnp.concatenate([W2t[0], W2t[1], W2t[2], W2t[3]], 0)
    W2_4p = jnp.concatenate([W2t[4], jnp.zeros((64, 128), _F32)], 0)
    b2 = p['enc_c2_b'][None]
    W3t = jnp.transpose(p['enc_c3_w'], (2, 1, 0))
    W3_01 = jnp.concatenate([W3t[0], W3t[1]], 0)
    W3_2p = jnp.concatenate([jnp.zeros((128, 256), _F32), W3t[2]], 0)
    b3 = p['enc_c3_b'][None]
    res_params = []
    for i in range(2):
        res_params += [
            p['res%d_gn1_g' % i][None], p['res%d_gn1_b' % i][None],
            jnp.transpose(p['res%d_c1_w' % i], (2, 1, 0)),
            p['res%d_gn2_g' % i][None], p['res%d_gn2_b' % i][None],
            p['res%d_c2_w' % i][:, :, 0].T,
        ]
    wpre = p['prevq_w'][:, :, 0].T
    bpre = p['prevq_b'][None]
    emb = p['codebook']
    embT = emb.T
    w_ct1 = p['dec_ct1_w']
    z256 = jnp.zeros((64, 256), _F32)
    P_ct1 = jnp.stack([
        jnp.concatenate([w_ct1[:, :, 1], w_ct1[:, :, 2]], 1),
        jnp.concatenate([z256, w_ct1[:, :, 0]], 1)])
    b_ct1 = jnp.concatenate([p['dec_ct1_b'], p['dec_ct1_b']])[None]
    w_ct2 = p['dec_ct2_w']
    z128b = jnp.zeros((256, 128), _F32)
    Q_ct2 = jnp.stack([
        jnp.concatenate([w_ct2[:, :, 4], z128b], 1),
        jnp.concatenate([w_ct2[:, :, 2], w_ct2[:, :, 3]], 1),
        jnp.concatenate([w_ct2[:, :, 0], w_ct2[:, :, 1]], 1)])
    b_ct2 = jnp.concatenate([p['dec_ct2_b'], p['dec_ct2_b']])[None]
    W7 = jnp.transpose(p['dec_c3_w'], (2, 1, 0))
    b7 = p['dec_c3_b'][None]

    h1 = _conv_stage(xt, W1, b1, (-3, -2, -1, 0, 1, 2, 3), relu=True)
    h2 = _enc2_stage(h1.reshape(B, 512, 128), W2_01, W2_4p, b2)
    z, idx3, counts = _vq_encoder_stage(
        h2.reshape(B, 256, 256), W3_01, W3_2p, b3, res_params, wpre, bpre,
        embT)
    emb_q = emb.astype(_BF16).astype(_F32)
    qflat = _sc_codebook_gather(emb_q, idx3.reshape(B * 256))
    q = qflat.reshape(B, 256, 64)
    y_pair, sq, pooled3 = _dec1_stage(q, z, P_ct1, b_ct1)
    y2 = _conv_stage(y_pair.reshape(B, 512, 256), Q_ct2, b_ct2, (-1, 0, 1),
                     relu=True)
    xrt = _conv_stage(y2.reshape(B, 1024, 128), W7, b7,
                      (-3, -2, -1, 0, 1, 2, 3), relu=False)
    x_recon = jnp.transpose(xrt, (0, 2, 1))
    loss11, perp11, behavior_pred = _head_stage(
        counts, sq, pooled3.reshape(B, 64),
        p['bh1_w'].T, p['bh1_b'][None], p['bh2_w'].T, p['bh2_b'][None],
        p['bh3_w'].T, p['bh3_b'][None],
        n_tok=float(B * 256), n_lat=float(B * 256 * 64))
    return (loss11.reshape(()), x_recon, perp11.reshape(()),
            behavior_pred)
